# baseline (speedup 1.0000x reference)
import math
from functools import partial

import jax
import jax.numpy as jnp
from jax.experimental import pallas as pl
from jax.experimental.pallas import tpu as pltpu

B, S, E, H = 4, 4096, 1024, 16
DH = E // H
SEL = max(1, int(5 * math.log(S + 1)))
SEL = min(SEL, S)
SELP = 48
ROWS = H * SELP
SB = 1024
NSB = S // SB
SCALE = 1.0 / math.sqrt(DH)
IDXW = 64


def _stats_kernel(x_ref, mean_ref, idx_ref, e_ref, cs_ref):
    s = pl.program_id(1)
    xb = x_ref[0]
    colsum = jnp.sum(xb, axis=0).reshape(1, E)
    energy = jnp.sum(xb * xb, axis=1).reshape(1, SB)

    @pl.when(s == 0)
    def _():
        cs_ref[...] = colsum

    @pl.when(s != 0)
    def _():
        cs_ref[...] += colsum

    e_ref[0, pl.ds(s * SB, SB)] = energy[0]

    @pl.when(s == NSB - 1)
    def _():
        mean_ref[0] = cs_ref[...] * (1.0 / S)
        lane_s = jax.lax.broadcasted_iota(jnp.int32, (1, S), 1)
        lane_w = jax.lax.broadcasted_iota(jnp.int32, (1, IDXW), 1)

        def body(j, carry):
            e, acc = carry
            m = jnp.max(e)
            idxv = jnp.min(jnp.where(e == m, lane_s, S))
            acc = jnp.where(lane_w == j, idxv, acc)
            e = jnp.where(lane_s == idxv, -1.0, e)
            return e, acc

        e0 = e_ref[...]
        _, acc = jax.lax.fori_loop(
            0, SEL, body, (e0, jnp.zeros((1, IDXW), jnp.int32)))
        idx_ref[0] = acc


def _stats(x):
    return pl.pallas_call(
        _stats_kernel,
        grid=(B, NSB),
        in_specs=[pl.BlockSpec((1, SB, E), lambda b, s: (b, s, 0))],
        out_specs=[
            pl.BlockSpec((1, 1, E), lambda b, s: (b, 0, 0)),
            pl.BlockSpec((1, 1, IDXW), lambda b, s: (b, 0, 0)),
        ],
        out_shape=[
            jax.ShapeDtypeStruct((B, 1, E), jnp.float32),
            jax.ShapeDtypeStruct((B, 1, IDXW), jnp.int32),
        ],
        scratch_shapes=[
            pltpu.VMEM((1, S), jnp.float32),
            pltpu.VMEM((1, E), jnp.float32),
        ],
    )(x)


def _gather_kernel(idx_ref, x_ref, out_ref):
    out_ref[...] = x_ref[...]


def _gather(x, idx2d):
    xr = x.reshape(B, S, 8, E // 8)
    out = pl.pallas_call(
        _gather_kernel,
        grid_spec=pltpu.PrefetchScalarGridSpec(
            num_scalar_prefetch=1,
            grid=(B, SELP),
            in_specs=[
                pl.BlockSpec((1, 1, 8, E // 8),
                             lambda b, i, idx: (b, idx[b, i], 0, 0)),
            ],
            out_specs=pl.BlockSpec((1, 1, 8, E // 8),
                                   lambda b, i, idx: (b, i, 0, 0)),
        ),
        out_shape=jax.ShapeDtypeStruct((B, SELP, 8, E // 8), jnp.float32),
    )(idx2d, xr)
    return out.reshape(B, SELP, E)


def _qtilde_kernel(xq_ref, wq_ref, wk_ref, bq_ref, qt_ref):
    q = jax.lax.dot_general(
        xq_ref[0], wq_ref[...], (((1,), (1,)), ((), ())),
        preferred_element_type=jnp.float32) + bq_ref[...]
    for h in range(H):
        qt_ref[0, h * SELP:(h + 1) * SELP, :] = jax.lax.dot_general(
            q[:, h * DH:(h + 1) * DH], wk_ref[h * DH:(h + 1) * DH, :],
            (((1,), (0,)), ((), ())), preferred_element_type=jnp.float32)


def _qtilde(xq, Wq, Wk, bq):
    return pl.pallas_call(
        _qtilde_kernel,
        grid=(B,),
        in_specs=[
            pl.BlockSpec((1, SELP, E), lambda b: (b, 0, 0)),
            pl.BlockSpec((E, E), lambda b: (0, 0)),
            pl.BlockSpec((E, E), lambda b: (0, 0)),
            pl.BlockSpec((1, E), lambda b: (0, 0)),
        ],
        out_specs=pl.BlockSpec((1, ROWS, E), lambda b: (b, 0, 0)),
        out_shape=jax.ShapeDtypeStruct((B, ROWS, E), jnp.float32),
    )(xq, Wq, Wk, bq)


def _flash_kernel(qt_ref, x_ref, z_ref, acc_ref, m_ref, l_ref):
    s = pl.program_id(1)
    xb = x_ref[0]
    scores = jax.lax.dot_general(
        qt_ref[0], xb, (((1,), (1,)), ((), ())),
        preferred_element_type=jnp.float32) * SCALE
    bmax = jnp.max(scores, axis=1, keepdims=True)

    @pl.when(s == 0)
    def _():
        p = jnp.exp(scores - bmax)
        m_ref[...] = bmax
        l_ref[...] = jnp.sum(p, axis=1, keepdims=True)
        acc_ref[...] = jax.lax.dot_general(
            p, xb, (((1,), (0,)), ((), ())),
            preferred_element_type=jnp.float32)

    @pl.when(s != 0)
    def _():
        m_old = m_ref[...]
        m_new = jnp.maximum(m_old, bmax)
        alpha = jnp.exp(m_old - m_new)
        p = jnp.exp(scores - m_new)
        m_ref[...] = m_new
        l_ref[...] = l_ref[...] * alpha + jnp.sum(p, axis=1, keepdims=True)
        acc_ref[...] = acc_ref[...] * alpha + jax.lax.dot_general(
            p, xb, (((1,), (0,)), ((), ())),
            preferred_element_type=jnp.float32)

    @pl.when(s == NSB - 1)
    def _():
        z_ref[0] = acc_ref[...] / l_ref[...]


def _flash(qt, x):
    return pl.pallas_call(
        _flash_kernel,
        grid=(B, NSB),
        in_specs=[
            pl.BlockSpec((1, ROWS, E), lambda b, s: (b, 0, 0)),
            pl.BlockSpec((1, SB, E), lambda b, s: (b, s, 0)),
        ],
        out_specs=pl.BlockSpec((1, ROWS, E), lambda b, s: (b, 0, 0)),
        out_shape=jax.ShapeDtypeStruct((B, ROWS, E), jnp.float32),
        scratch_shapes=[
            pltpu.VMEM((ROWS, E), jnp.float32),
            pltpu.VMEM((ROWS, 1), jnp.float32),
            pltpu.VMEM((ROWS, 1), jnp.float32),
        ],
    )(qt, x)


def _epilogue_kernel(z_ref, wv_ref, wo_ref, bv_ref, bo_ref, out_ref):
    acc = jax.lax.dot_general(
        bv_ref[...], wo_ref[...], (((1,), (1,)), ((), ())),
        preferred_element_type=jnp.float32) + bo_ref[...]
    acc = jnp.broadcast_to(acc, (SELP, E))
    for h in range(H):
        oh = jax.lax.dot_general(
            z_ref[0, h * SELP:(h + 1) * SELP, :], wv_ref[h * DH:(h + 1) * DH, :],
            (((1,), (1,)), ((), ())), preferred_element_type=jnp.float32)
        acc = acc + jax.lax.dot_general(
            oh, wo_ref[:, h * DH:(h + 1) * DH],
            (((1,), (1,)), ((), ())), preferred_element_type=jnp.float32)
    out_ref[0] = acc


def _epilogue(z, Wv, Wo, bv, bo):
    return pl.pallas_call(
        _epilogue_kernel,
        grid=(B,),
        in_specs=[
            pl.BlockSpec((1, ROWS, E), lambda b: (b, 0, 0)),
            pl.BlockSpec((E, E), lambda b: (0, 0)),
            pl.BlockSpec((E, E), lambda b: (0, 0)),
            pl.BlockSpec((1, E), lambda b: (0, 0)),
            pl.BlockSpec((1, E), lambda b: (0, 0)),
        ],
        out_specs=pl.BlockSpec((1, SELP, E), lambda b: (b, 0, 0)),
        out_shape=jax.ShapeDtypeStruct((B, SELP, E), jnp.float32),
    )(z, Wv, Wo, bv, bo)


def _assemble_kernel(mean_ref, rows_ref, idx_ref, out_ref):
    s = pl.program_id(1)
    mean = mean_ref[0]
    delta = rows_ref[0] - mean
    row_g = jax.lax.broadcasted_iota(jnp.int32, (SB, SELP), 0) + s * SB
    col_j = jax.lax.broadcasted_iota(jnp.int32, (SB, SELP), 1)
    idxv = idx_ref[0, 0, :SELP].reshape(1, SELP)
    onehot = ((row_g == idxv) & (col_j < SEL)).astype(jnp.float32)
    out_ref[0] = jnp.broadcast_to(mean, (SB, E)) + jax.lax.dot_general(
        onehot, delta, (((1,), (0,)), ((), ())),
        preferred_element_type=jnp.float32)


def _assemble(mean, rows, idx):
    return pl.pallas_call(
        _assemble_kernel,
        grid=(B, NSB),
        in_specs=[
            pl.BlockSpec((1, 1, E), lambda b, s: (b, 0, 0)),
            pl.BlockSpec((1, SELP, E), lambda b, s: (b, 0, 0)),
            pl.BlockSpec((1, 1, IDXW), lambda b, s: (b, 0, 0)),
        ],
        out_specs=pl.BlockSpec((1, SB, E), lambda b, s: (b, s, 0)),
        out_shape=jax.ShapeDtypeStruct((B, S, E), jnp.float32),
    )(mean, rows, idx)


def kernel(x, W_qkv, b_qkv, W_out, b_out):
    Wq = W_qkv[0:E]
    Wk = W_qkv[E:2 * E]
    Wv = W_qkv[2 * E:3 * E]
    bq = b_qkv[0:E].reshape(1, E)
    bv = b_qkv[2 * E:3 * E].reshape(1, E)
    bo = b_out.reshape(1, E)

    mean, idx = _stats(x)
    xq = _gather(x, idx.reshape(B, IDXW))
    qt = _qtilde(xq, Wq, Wk, bq)
    z = _flash(qt, x)
    rows = _epilogue(z, Wv, W_out, bv, bo)
    return _assemble(mean, rows, idx)


# speedup vs baseline: 1.4745x; 1.4745x over previous
import math
from functools import partial

import jax
import jax.numpy as jnp
from jax.experimental import pallas as pl
from jax.experimental.pallas import tpu as pltpu

B, S, E, H = 4, 4096, 1024, 16
DH = E // H
SEL = max(1, int(5 * math.log(S + 1)))
SEL = min(SEL, S)
SELP = 48
ROWS = H * SELP
SB = 1024
NSB = S // SB
SCALE = 1.0 / math.sqrt(DH)
IDXW = 64


def _stats_kernel(x_ref, mean_ref, idx_ref, e_ref, cs_ref):
    s = pl.program_id(1)
    xb = x_ref[0]
    colsum = jnp.sum(xb, axis=0).reshape(1, E)
    energy = jnp.sum(xb * xb, axis=1).reshape(1, SB)

    @pl.when(s == 0)
    def _():
        cs_ref[...] = colsum

    @pl.when(s != 0)
    def _():
        cs_ref[...] += colsum

    e_ref[0, pl.ds(s * SB, SB)] = energy[0]

    @pl.when(s == NSB - 1)
    def _():
        mean_ref[0] = cs_ref[...] * (1.0 / S)
        lane_s = jax.lax.broadcasted_iota(jnp.int32, (1, S), 1)
        lane_w = jax.lax.broadcasted_iota(jnp.int32, (1, IDXW), 1)

        def body(j, carry):
            e, acc = carry
            m = jnp.max(e)
            idxv = jnp.min(jnp.where(e == m, lane_s, S))
            acc = jnp.where(lane_w == j, idxv, acc)
            e = jnp.where(lane_s == idxv, -1.0, e)
            return e, acc

        e0 = e_ref[...]
        _, acc = jax.lax.fori_loop(
            0, SEL, body, (e0, jnp.zeros((1, IDXW), jnp.int32)))
        idx_ref[0] = acc


def _stats(x):
    return pl.pallas_call(
        _stats_kernel,
        grid=(B, NSB),
        in_specs=[pl.BlockSpec((1, SB, E), lambda b, s: (b, s, 0))],
        out_specs=[
            pl.BlockSpec((1, 1, E), lambda b, s: (b, 0, 0)),
            pl.BlockSpec((1, 1, IDXW), lambda b, s: (b, 0, 0)),
        ],
        out_shape=[
            jax.ShapeDtypeStruct((B, 1, E), jnp.float32),
            jax.ShapeDtypeStruct((B, 1, IDXW), jnp.int32),
        ],
        scratch_shapes=[
            pltpu.VMEM((1, S), jnp.float32),
            pltpu.VMEM((1, E), jnp.float32),
        ],
    )(x)


def _qtilde_kernel(idx_ref, x_ref, wq_ref, wk_ref, bq_ref, qt_ref,
                   xq_ref, sem):
    b = pl.program_id(0)
    copies = []
    for i in range(SELP):
        r = idx_ref[b, i]
        c = pltpu.make_async_copy(
            x_ref.at[b, pl.ds(r, 1), :], xq_ref.at[pl.ds(i, 1), :], sem)
        c.start()
        copies.append(c)
    for c in copies:
        c.wait()
    q = jax.lax.dot_general(
        xq_ref[...].astype(jnp.bfloat16), wq_ref[...],
        (((1,), (1,)), ((), ())),
        preferred_element_type=jnp.float32) + bq_ref[...]
    q16 = q.astype(jnp.bfloat16)
    for h in range(H):
        qt_ref[0, h * SELP:(h + 1) * SELP, :] = jax.lax.dot_general(
            q16[:, h * DH:(h + 1) * DH], wk_ref[h * DH:(h + 1) * DH, :],
            (((1,), (0,)), ((), ())),
            preferred_element_type=jnp.float32).astype(jnp.bfloat16)


def _qtilde(x, idx2d, Wq16, Wk16, bq):
    return pl.pallas_call(
        _qtilde_kernel,
        grid_spec=pltpu.PrefetchScalarGridSpec(
            num_scalar_prefetch=1,
            grid=(B,),
            in_specs=[
                pl.BlockSpec(memory_space=pl.ANY),
                pl.BlockSpec((E, E), lambda b, idx: (0, 0)),
                pl.BlockSpec((E, E), lambda b, idx: (0, 0)),
                pl.BlockSpec((1, E), lambda b, idx: (0, 0)),
            ],
            out_specs=pl.BlockSpec((1, ROWS, E), lambda b, idx: (b, 0, 0)),
            scratch_shapes=[
                pltpu.VMEM((SELP, E), jnp.float32),
                pltpu.SemaphoreType.DMA,
            ],
        ),
        out_shape=jax.ShapeDtypeStruct((B, ROWS, E), jnp.bfloat16),
    )(idx2d, x, Wq16, Wk16, bq)


def _flash_kernel(qt_ref, x_ref, z_ref, acc_ref, m_ref, l_ref):
    s = pl.program_id(1)
    xb16 = x_ref[0].astype(jnp.bfloat16)
    scores = jax.lax.dot_general(
        qt_ref[0], xb16, (((1,), (1,)), ((), ())),
        preferred_element_type=jnp.float32) * SCALE
    bmax = jnp.max(scores, axis=1, keepdims=True)

    @pl.when(s == 0)
    def _():
        p = jnp.exp(scores - bmax)
        m_ref[...] = bmax
        l_ref[...] = jnp.sum(p, axis=1, keepdims=True)
        acc_ref[...] = jax.lax.dot_general(
            p.astype(jnp.bfloat16), xb16, (((1,), (0,)), ((), ())),
            preferred_element_type=jnp.float32)

    @pl.when(s != 0)
    def _():
        m_old = m_ref[...]
        m_new = jnp.maximum(m_old, bmax)
        alpha = jnp.exp(m_old - m_new)
        p = jnp.exp(scores - m_new)
        m_ref[...] = m_new
        l_ref[...] = l_ref[...] * alpha + jnp.sum(p, axis=1, keepdims=True)
        acc_ref[...] = acc_ref[...] * alpha + jax.lax.dot_general(
            p.astype(jnp.bfloat16), xb16, (((1,), (0,)), ((), ())),
            preferred_element_type=jnp.float32)

    @pl.when(s == NSB - 1)
    def _():
        z_ref[0] = acc_ref[...] / l_ref[...]


def _flash(qt, x):
    return pl.pallas_call(
        _flash_kernel,
        grid=(B, NSB),
        in_specs=[
            pl.BlockSpec((1, ROWS, E), lambda b, s: (b, 0, 0)),
            pl.BlockSpec((1, SB, E), lambda b, s: (b, s, 0)),
        ],
        out_specs=pl.BlockSpec((1, ROWS, E), lambda b, s: (b, 0, 0)),
        out_shape=jax.ShapeDtypeStruct((B, ROWS, E), jnp.float32),
        compiler_params=pltpu.CompilerParams(
            dimension_semantics=("parallel", "arbitrary")),
        scratch_shapes=[
            pltpu.VMEM((ROWS, E), jnp.float32),
            pltpu.VMEM((ROWS, 1), jnp.float32),
            pltpu.VMEM((ROWS, 1), jnp.float32),
        ],
    )(qt, x)


def _epilogue_kernel(z_ref, wv_ref, wo_ref, bv_ref, bo_ref, out_ref):
    bvo = jax.lax.dot_general(
        bv_ref[...].astype(jnp.bfloat16), wo_ref[...],
        (((1,), (1,)), ((), ())),
        preferred_element_type=jnp.float32) + bo_ref[...]
    acc = jnp.broadcast_to(bvo, (SELP, E))
    for h in range(H):
        oh = jax.lax.dot_general(
            z_ref[0, h * SELP:(h + 1) * SELP, :].astype(jnp.bfloat16),
            wv_ref[h * DH:(h + 1) * DH, :],
            (((1,), (1,)), ((), ())), preferred_element_type=jnp.float32)
        acc = acc + jax.lax.dot_general(
            oh.astype(jnp.bfloat16), wo_ref[:, h * DH:(h + 1) * DH],
            (((1,), (1,)), ((), ())), preferred_element_type=jnp.float32)
    out_ref[0] = acc


def _epilogue(z, Wv, Wo, bv, bo):
    return pl.pallas_call(
        _epilogue_kernel,
        grid=(B,),
        in_specs=[
            pl.BlockSpec((1, ROWS, E), lambda b: (b, 0, 0)),
            pl.BlockSpec((E, E), lambda b: (0, 0)),
            pl.BlockSpec((E, E), lambda b: (0, 0)),
            pl.BlockSpec((1, E), lambda b: (0, 0)),
            pl.BlockSpec((1, E), lambda b: (0, 0)),
        ],
        out_specs=pl.BlockSpec((1, SELP, E), lambda b: (b, 0, 0)),
        out_shape=jax.ShapeDtypeStruct((B, SELP, E), jnp.float32),
    )(z, Wv, Wo, bv, bo)


def _assemble_kernel(mean_ref, rows_ref, idx_ref, out_ref):
    s = pl.program_id(1)
    mean = mean_ref[0]
    delta = rows_ref[0] - mean
    row_g = jax.lax.broadcasted_iota(jnp.int32, (SB, SELP), 0) + s * SB
    col_j = jax.lax.broadcasted_iota(jnp.int32, (SB, SELP), 1)
    idxv = idx_ref[0, 0, :SELP].reshape(1, SELP)
    onehot = ((row_g == idxv) & (col_j < SEL)).astype(jnp.bfloat16)
    out_ref[0] = jnp.broadcast_to(mean, (SB, E)) + jax.lax.dot_general(
        onehot, delta.astype(jnp.bfloat16), (((1,), (0,)), ((), ())),
        preferred_element_type=jnp.float32)


def _assemble(mean, rows, idx):
    return pl.pallas_call(
        _assemble_kernel,
        grid=(B, NSB),
        in_specs=[
            pl.BlockSpec((1, 1, E), lambda b, s: (b, 0, 0)),
            pl.BlockSpec((1, SELP, E), lambda b, s: (b, 0, 0)),
            pl.BlockSpec((1, 1, IDXW), lambda b, s: (b, 0, 0)),
        ],
        out_specs=pl.BlockSpec((1, SB, E), lambda b, s: (b, s, 0)),
        out_shape=jax.ShapeDtypeStruct((B, S, E), jnp.float32),
    )(mean, rows, idx)


def kernel(x, W_qkv, b_qkv, W_out, b_out):
    Wq16 = W_qkv[0:E].astype(jnp.bfloat16)
    Wk16 = W_qkv[E:2 * E].astype(jnp.bfloat16)
    Wv16 = W_qkv[2 * E:3 * E].astype(jnp.bfloat16)
    Wo16 = W_out.astype(jnp.bfloat16)
    bq = b_qkv[0:E].reshape(1, E)
    bv = b_qkv[2 * E:3 * E].reshape(1, E)
    bo = b_out.reshape(1, E)

    mean, idx = _stats(x)
    qt = _qtilde(x, idx.reshape(B, IDXW), Wq16, Wk16, bq)
    z = _flash(qt, x)
    rows = _epilogue(z, Wv16, Wo16, bv, bo)
    return _assemble(mean, rows, idx)


# speedup vs baseline: 1.7949x; 1.2173x over previous
import math
from functools import partial

import jax
import jax.numpy as jnp
from jax.experimental import pallas as pl
from jax.experimental.pallas import tpu as pltpu

B, S, E, H = 4, 4096, 1024, 16
DH = E // H
SEL = max(1, int(5 * math.log(S + 1)))
SEL = min(SEL, S)
SELP = 48
ROWS = H * SELP
SB = 1024
NSB = S // SB
SCALE = 1.0 / math.sqrt(DH)
IDXW = 64
BF = jnp.bfloat16


def _head_mask(rows, cols, row_group, col_group, dtype):
    r = jax.lax.broadcasted_iota(jnp.int32, (rows, cols), 0) // row_group
    c = jax.lax.broadcasted_iota(jnp.int32, (rows, cols), 1) // col_group
    return (r == c).astype(dtype)


def _stats_kernel(x_ref, mean_ref, idx_ref, e_ref, cs_ref):
    b = pl.program_id(0)
    s = pl.program_id(1)
    xb = x_ref[0]
    colsum = jnp.sum(xb, axis=0).reshape(1, E)
    energy = jnp.sum(xb * xb, axis=1).reshape(1, SB)

    @pl.when(s == 0)
    def _():
        cs_ref[...] = colsum

    @pl.when(s != 0)
    def _():
        cs_ref[...] += colsum

    e_ref[pl.ds(b, 1), pl.ds(s * SB, SB)] = energy

    @pl.when(s == NSB - 1)
    def _():
        mean_ref[0] = cs_ref[...] * (1.0 / S)

    @pl.when((b == B - 1) & (s == NSB - 1))
    def _():
        lane_s = jax.lax.broadcasted_iota(jnp.int32, (B, S), 1)
        lane_w = jax.lax.broadcasted_iota(jnp.int32, (B, IDXW), 1)

        def body(j, carry):
            e, acc = carry
            m = jnp.max(e, axis=1, keepdims=True)
            idxv = jnp.min(jnp.where(e == m, lane_s, S), axis=1, keepdims=True)
            acc = jnp.where(lane_w == j, idxv, acc)
            e = jnp.where(lane_s == idxv, -1.0, e)
            return e, acc

        _, acc = jax.lax.fori_loop(
            0, SEL, body, (e_ref[...], jnp.zeros((B, IDXW), jnp.int32)))
        idx_ref[...] = acc.reshape(B, 1, IDXW)


def _stats(x):
    return pl.pallas_call(
        _stats_kernel,
        grid=(B, NSB),
        in_specs=[pl.BlockSpec((1, SB, E), lambda b, s: (b, s, 0))],
        out_specs=[
            pl.BlockSpec((1, 1, E), lambda b, s: (b, 0, 0)),
            pl.BlockSpec((B, 1, IDXW), lambda b, s: (0, 0, 0)),
        ],
        out_shape=[
            jax.ShapeDtypeStruct((B, 1, E), jnp.float32),
            jax.ShapeDtypeStruct((B, 1, IDXW), jnp.int32),
        ],
        scratch_shapes=[
            pltpu.VMEM((B, S), jnp.float32),
            pltpu.VMEM((1, E), jnp.float32),
        ],
    )(x)


def _qtilde_kernel(idx_ref, x_ref, wq_ref, wk_ref, bq_ref, qt_ref,
                   xq_ref, sem):
    b = pl.program_id(0)
    copies = []
    for i in range(SELP):
        r = idx_ref[b, i]
        c = pltpu.make_async_copy(
            x_ref.at[b, pl.ds(r, 1), :], xq_ref.at[pl.ds(i, 1), :], sem)
        c.start()
        copies.append(c)
    for c in copies:
        c.wait()
    q = jax.lax.dot_general(
        xq_ref[...].astype(BF), wq_ref[...], (((1,), (1,)), ((), ())),
        preferred_element_type=jnp.float32) + bq_ref[...]
    qe = jnp.concatenate([q] * H, axis=0)
    qe = (qe * _head_mask(ROWS, E, SELP, DH, jnp.float32)).astype(BF)
    qt_ref[0] = jax.lax.dot_general(
        qe, wk_ref[...], (((1,), (0,)), ((), ())),
        preferred_element_type=jnp.float32).astype(BF)


def _qtilde(x, idx2d, Wq16, Wk16, bq):
    return pl.pallas_call(
        _qtilde_kernel,
        grid_spec=pltpu.PrefetchScalarGridSpec(
            num_scalar_prefetch=1,
            grid=(B,),
            in_specs=[
                pl.BlockSpec(memory_space=pl.ANY),
                pl.BlockSpec((E, E), lambda b, idx: (0, 0)),
                pl.BlockSpec((E, E), lambda b, idx: (0, 0)),
                pl.BlockSpec((1, E), lambda b, idx: (0, 0)),
            ],
            out_specs=pl.BlockSpec((1, ROWS, E), lambda b, idx: (b, 0, 0)),
            scratch_shapes=[
                pltpu.VMEM((SELP, E), jnp.float32),
                pltpu.SemaphoreType.DMA,
            ],
        ),
        out_shape=jax.ShapeDtypeStruct((B, ROWS, E), BF),
    )(idx2d, x, Wq16, Wk16, bq)


def _flash_kernel(qt_ref, x_ref, mean_ref, wv_ref, wo_ref, bv_ref, bo_ref,
                  fill_ref, rows_ref, acc_ref, m_ref, l_ref):
    s = pl.program_id(1)
    mean = mean_ref[0]
    fill_ref[0] = jnp.broadcast_to(mean, (SB, E))

    xb16 = x_ref[0].astype(BF)
    scores = jax.lax.dot_general(
        qt_ref[0], xb16, (((1,), (1,)), ((), ())),
        preferred_element_type=jnp.float32) * SCALE
    bmax = jnp.max(scores, axis=1, keepdims=True)

    @pl.when(s == 0)
    def _():
        p = jnp.exp(scores - bmax)
        m_ref[...] = bmax
        l_ref[...] = jnp.sum(p, axis=1, keepdims=True)
        acc_ref[...] = jax.lax.dot_general(
            p.astype(BF), xb16, (((1,), (0,)), ((), ())),
            preferred_element_type=jnp.float32)

    @pl.when(s != 0)
    def _():
        m_old = m_ref[...]
        m_new = jnp.maximum(m_old, bmax)
        alpha = jnp.exp(m_old - m_new)
        p = jnp.exp(scores - m_new)
        m_ref[...] = m_new
        l_ref[...] = l_ref[...] * alpha + jnp.sum(p, axis=1, keepdims=True)
        acc_ref[...] = acc_ref[...] * alpha + jax.lax.dot_general(
            p.astype(BF), xb16, (((1,), (0,)), ((), ())),
            preferred_element_type=jnp.float32)

    @pl.when(s == NSB - 1)
    def _():
        z = acc_ref[...] / l_ref[...]
        oh = jax.lax.dot_general(
            z.astype(BF), wv_ref[...], (((1,), (1,)), ((), ())),
            preferred_element_type=jnp.float32)
        oh = oh * _head_mask(ROWS, E, SELP, DH, jnp.float32)
        folded = jnp.zeros((SELP, E), jnp.float32)
        for h in range(H):
            folded = folded + oh[h * SELP:(h + 1) * SELP, :]
        bvo = jax.lax.dot_general(
            bv_ref[...].astype(BF), wo_ref[...], (((1,), (1,)), ((), ())),
            preferred_element_type=jnp.float32) + bo_ref[...]
        rows_ref[0] = jax.lax.dot_general(
            folded.astype(BF), wo_ref[...], (((1,), (1,)), ((), ())),
            preferred_element_type=jnp.float32) + bvo


def _flash(qt, x, mean, Wv16, Wo16, bv, bo):
    return pl.pallas_call(
        _flash_kernel,
        grid=(B, NSB),
        in_specs=[
            pl.BlockSpec((1, ROWS, E), lambda b, s: (b, 0, 0)),
            pl.BlockSpec((1, SB, E), lambda b, s: (b, s, 0)),
            pl.BlockSpec((1, 1, E), lambda b, s: (b, 0, 0)),
            pl.BlockSpec((E, E), lambda b, s: (0, 0)),
            pl.BlockSpec((E, E), lambda b, s: (0, 0)),
            pl.BlockSpec((1, E), lambda b, s: (0, 0)),
            pl.BlockSpec((1, E), lambda b, s: (0, 0)),
        ],
        out_specs=[
            pl.BlockSpec((1, SB, E), lambda b, s: (b, s, 0)),
            pl.BlockSpec((1, SELP, E), lambda b, s: (b, 0, 0)),
        ],
        out_shape=[
            jax.ShapeDtypeStruct((B, S, E), jnp.float32),
            jax.ShapeDtypeStruct((B, SELP, E), jnp.float32),
        ],
        scratch_shapes=[
            pltpu.VMEM((ROWS, E), jnp.float32),
            pltpu.VMEM((ROWS, 1), jnp.float32),
            pltpu.VMEM((ROWS, 1), jnp.float32),
        ],
        compiler_params=pltpu.CompilerParams(
            dimension_semantics=("parallel", "arbitrary")),
    )(qt, x, mean, Wv16, Wo16, bv, bo)


def _scatter_kernel(idx_ref, fill_ref, rows_ref, out_ref, sem):
    b = pl.program_id(0)
    copies = []
    for i in range(SEL):
        r = idx_ref[b, i]
        c = pltpu.make_async_copy(
            rows_ref.at[b, pl.ds(i, 1), :], out_ref.at[b, pl.ds(r, 1), :], sem)
        c.start()
        copies.append(c)
    for c in copies:
        c.wait()


def _scatter(fill, rows, idx2d):
    return pl.pallas_call(
        _scatter_kernel,
        grid_spec=pltpu.PrefetchScalarGridSpec(
            num_scalar_prefetch=1,
            grid=(B,),
            in_specs=[
                pl.BlockSpec(memory_space=pl.ANY),
                pl.BlockSpec(memory_space=pl.ANY),
            ],
            out_specs=pl.BlockSpec(memory_space=pl.ANY),
            scratch_shapes=[pltpu.SemaphoreType.DMA],
        ),
        out_shape=jax.ShapeDtypeStruct((B, S, E), jnp.float32),
        input_output_aliases={1: 0},
    )(idx2d, fill, rows)


def kernel(x, W_qkv, b_qkv, W_out, b_out):
    Wq16 = W_qkv[0:E].astype(BF)
    Wk16 = W_qkv[E:2 * E].astype(BF)
    Wv16 = W_qkv[2 * E:3 * E].astype(BF)
    Wo16 = W_out.astype(BF)
    bq = b_qkv[0:E].reshape(1, E)
    bv = b_qkv[2 * E:3 * E].reshape(1, E)
    bo = b_out.reshape(1, E)

    mean, idx = _stats(x)
    idx2d = idx.reshape(B, IDXW)
    qt = _qtilde(x, idx2d, Wq16, Wk16, bq)
    fill, rows = _flash(qt, x, mean, Wv16, Wo16, bv, bo)
    return _scatter(fill, rows, idx2d)


# speedup vs baseline: 1.9687x; 1.0968x over previous
import math
from functools import partial

import jax
import jax.numpy as jnp
from jax.experimental import pallas as pl
from jax.experimental.pallas import tpu as pltpu

B, S, E, H = 4, 4096, 1024, 16
DH = E // H
SEL = max(1, int(5 * math.log(S + 1)))
SEL = min(SEL, S)
SELP = 48
ROWS = H * SELP
SB = 1024
NSB = S // SB
T_STEPS = B * NSB + NSB
SCALE = 1.0 / math.sqrt(DH)
IDXW = 64
BF = jnp.bfloat16


def _head_mask(rows, cols, row_group, col_group, dtype):
    r = jax.lax.broadcasted_iota(jnp.int32, (rows, cols), 0) // row_group
    c = jax.lax.broadcasted_iota(jnp.int32, (rows, cols), 1) // col_group
    return (r == c).astype(dtype)


def _stats_kernel(x_ref, mean_ref, idx_ref, e_ref, cs_ref):
    b = pl.program_id(0)
    s = pl.program_id(1)
    xb = x_ref[0]
    colsum = jnp.sum(xb, axis=0).reshape(1, E)
    energy = jnp.sum(xb * xb, axis=1).reshape(1, SB)

    @pl.when(s == 0)
    def _():
        cs_ref[...] = colsum

    @pl.when(s != 0)
    def _():
        cs_ref[...] += colsum

    e_ref[pl.ds(b, 1), pl.ds(s * SB, SB)] = energy

    @pl.when(s == NSB - 1)
    def _():
        mean_ref[0] = cs_ref[...] * (1.0 / S)

    @pl.when((b == B - 1) & (s == NSB - 1))
    def _():
        lane_s = jax.lax.broadcasted_iota(jnp.int32, (B, S), 1)
        lane_w = jax.lax.broadcasted_iota(jnp.int32, (B, IDXW), 1)

        def body(j, carry):
            e, acc = carry
            m = jnp.max(e, axis=1, keepdims=True)
            idxv = jnp.min(jnp.where(e == m, lane_s, S), axis=1, keepdims=True)
            acc = jnp.where(lane_w == j, idxv, acc)
            e = jnp.where(lane_s == idxv, -1.0, e)
            return e, acc

        _, acc = jax.lax.fori_loop(
            0, SEL, body, (e_ref[...], jnp.zeros((B, IDXW), jnp.int32)))
        idx_ref[...] = acc.reshape(B, 1, IDXW)


def _stats(x):
    return pl.pallas_call(
        _stats_kernel,
        grid=(B, NSB),
        in_specs=[pl.BlockSpec((1, SB, E), lambda b, s: (b, s, 0))],
        out_specs=[
            pl.BlockSpec((1, 1, E), lambda b, s: (b, 0, 0)),
            pl.BlockSpec((B, 1, IDXW), lambda b, s: (0, 0, 0)),
        ],
        out_shape=[
            jax.ShapeDtypeStruct((B, 1, E), jnp.float32),
            jax.ShapeDtypeStruct((B, 1, IDXW), jnp.int32),
        ],
        scratch_shapes=[
            pltpu.VMEM((B, S), jnp.float32),
            pltpu.VMEM((1, E), jnp.float32),
        ],
    )(x)


def _bc(t):
    return jnp.minimum(t // NSB, B - 1)


def _bf(t):
    return jnp.maximum(t - NSB, 0) // NSB


def _sf(t):
    return jnp.maximum(t - NSB, 0) % NSB


def _mega_kernel(idxp_ref, xany_ref, x_ref, mean_ref, idx_ref,
                 wq_ref, wk_ref, wv_ref, wo_ref, bq_ref, bv_ref, bo_ref,
                 fill_ref,
                 qt_ref, xq_ref, rows_ref, acc_ref, m_ref, l_ref, sem):
    t = pl.program_id(0)
    bc = _bc(t)
    s = t % NSB
    compute = t < B * NSB

    @pl.when(t >= NSB)
    def _():
        bf = _bf(t)
        sf = _sf(t)
        mean = mean_ref[0]
        delta = rows_ref[...] - mean
        row_g = jax.lax.broadcasted_iota(jnp.int32, (SB, SELP), 0) + sf * SB
        col_j = jax.lax.broadcasted_iota(jnp.int32, (SB, SELP), 1)
        idxv = idx_ref[bf, 0, :SELP].reshape(1, SELP)
        onehot = ((row_g == idxv) & (col_j < SEL)).astype(BF)
        fill_ref[0] = jnp.broadcast_to(mean, (SB, E)) + jax.lax.dot_general(
            onehot, delta.astype(BF), (((1,), (0,)), ((), ())),
            preferred_element_type=jnp.float32)

    @pl.when(compute & (s == 0))
    def _():
        copies = []
        for i in range(SELP):
            r = idxp_ref[bc, i]
            c = pltpu.make_async_copy(
                xany_ref.at[bc, pl.ds(r, 1), :], xq_ref.at[pl.ds(i, 1), :],
                sem)
            c.start()
            copies.append(c)
        for c in copies:
            c.wait()
        q = jax.lax.dot_general(
            xq_ref[...].astype(BF), wq_ref[...], (((1,), (1,)), ((), ())),
            preferred_element_type=jnp.float32) + bq_ref[...]
        qe = jnp.concatenate([q] * H, axis=0)
        qe = (qe * _head_mask(ROWS, E, SELP, DH, jnp.float32)).astype(BF)
        qt = jax.lax.dot_general(
            qe, wk_ref[...], (((1,), (0,)), ((), ())),
            preferred_element_type=jnp.float32)
        qt_ref[...] = (qt * SCALE).astype(BF)

    @pl.when(compute)
    def _():
        xb16 = x_ref[0].astype(BF)
        scores = jax.lax.dot_general(
            qt_ref[...], xb16, (((1,), (1,)), ((), ())),
            preferred_element_type=jnp.float32)
        bmax = jnp.max(scores, axis=1, keepdims=True)

        @pl.when(s == 0)
        def _():
            p = jnp.exp(scores - bmax)
            m_ref[...] = bmax
            l_ref[...] = jnp.sum(p, axis=1, keepdims=True)
            acc_ref[...] = jax.lax.dot_general(
                p.astype(BF), xb16, (((1,), (0,)), ((), ())),
                preferred_element_type=jnp.float32)

        @pl.when(s != 0)
        def _():
            m_old = m_ref[...]
            m_new = jnp.maximum(m_old, bmax)
            alpha = jnp.exp(m_old - m_new)
            p = jnp.exp(scores - m_new)
            m_ref[...] = m_new
            l_ref[...] = l_ref[...] * alpha + jnp.sum(p, axis=1, keepdims=True)
            acc_ref[...] = acc_ref[...] * alpha + jax.lax.dot_general(
                p.astype(BF), xb16, (((1,), (0,)), ((), ())),
                preferred_element_type=jnp.float32)

        @pl.when(s == NSB - 1)
        def _():
            z = acc_ref[...] / l_ref[...]
            oh = jax.lax.dot_general(
                z.astype(BF), wv_ref[...], (((1,), (1,)), ((), ())),
                preferred_element_type=jnp.float32)
            oh = oh * _head_mask(ROWS, E, SELP, DH, jnp.float32)
            folded = jnp.zeros((SELP, E), jnp.float32)
            for h in range(H):
                folded = folded + oh[h * SELP:(h + 1) * SELP, :]
            bvo = jax.lax.dot_general(
                bv_ref[...].astype(BF), wo_ref[...], (((1,), (1,)), ((), ())),
                preferred_element_type=jnp.float32) + bo_ref[...]
            rows_ref[...] = jax.lax.dot_general(
                folded.astype(BF), wo_ref[...], (((1,), (1,)), ((), ())),
                preferred_element_type=jnp.float32) + bvo


def _mega(x, idx2d, mean, idx, Wq16, Wk16, Wv16, Wo16, bq, bv, bo):
    wspec = pl.BlockSpec((E, E), lambda t, p: (0, 0))
    bspec = pl.BlockSpec((1, E), lambda t, p: (0, 0))
    return pl.pallas_call(
        _mega_kernel,
        grid_spec=pltpu.PrefetchScalarGridSpec(
            num_scalar_prefetch=1,
            grid=(T_STEPS,),
            in_specs=[
                pl.BlockSpec(memory_space=pl.ANY),
                pl.BlockSpec((1, SB, E),
                             lambda t, p: (_bc(t),
                                           jnp.minimum(t, B * NSB - 1) % NSB,
                                           0)),
                pl.BlockSpec((1, 1, E), lambda t, p: (_bf(t), 0, 0)),
                pl.BlockSpec((B, 1, IDXW), lambda t, p: (0, 0, 0)),
                wspec, wspec, wspec, wspec,
                bspec, bspec, bspec,
            ],
            out_specs=pl.BlockSpec(
                (1, SB, E), lambda t, p: (_bf(t), _sf(t), 0)),
            scratch_shapes=[
                pltpu.VMEM((ROWS, E), BF),
                pltpu.VMEM((SELP, E), jnp.float32),
                pltpu.VMEM((SELP, E), jnp.float32),
                pltpu.VMEM((ROWS, E), jnp.float32),
                pltpu.VMEM((ROWS, 1), jnp.float32),
                pltpu.VMEM((ROWS, 1), jnp.float32),
                pltpu.SemaphoreType.DMA,
            ],
        ),
        out_shape=jax.ShapeDtypeStruct((B, S, E), jnp.float32),
    )(idx2d, x, x, mean, idx, Wq16, Wk16, Wv16, Wo16, bq, bv, bo)


def kernel(x, W_qkv, b_qkv, W_out, b_out):
    Wq16 = W_qkv[0:E].astype(BF)
    Wk16 = W_qkv[E:2 * E].astype(BF)
    Wv16 = W_qkv[2 * E:3 * E].astype(BF)
    Wo16 = W_out.astype(BF)
    bq = b_qkv[0:E].reshape(1, E)
    bv = b_qkv[2 * E:3 * E].reshape(1, E)
    bo = b_out.reshape(1, E)

    mean, idx = _stats(x)
    idx2d = idx.reshape(B, IDXW)
    return _mega(x, idx2d, mean, idx, Wq16, Wk16, Wv16, Wo16, bq, bv, bo)


# speedup vs baseline: 2.0839x; 1.0585x over previous
import math
from functools import partial

import jax
import jax.numpy as jnp
from jax.experimental import pallas as pl
from jax.experimental.pallas import tpu as pltpu

B, S, E, H = 4, 4096, 1024, 16
DH = E // H
SEL = max(1, int(5 * math.log(S + 1)))
SEL = min(SEL, S)
SELP = 48
ROWS = H * SELP
SB = 1024
NSB = S // SB
T_STEPS = B * NSB + NSB
SCALE = 1.0 / math.sqrt(DH)
IDXW = 64
BF = jnp.bfloat16


def _head_mask(rows, cols, row_group, col_group, dtype):
    r = jax.lax.broadcasted_iota(jnp.int32, (rows, cols), 0) // row_group
    c = jax.lax.broadcasted_iota(jnp.int32, (rows, cols), 1) // col_group
    return (r == c).astype(dtype)


def _stats_kernel(x_ref, mean_ref, idx_ref, e_ref, cs_ref):
    b = pl.program_id(0)
    s = pl.program_id(1)
    xb = x_ref[0]
    colsum = jnp.sum(xb, axis=0).reshape(1, E)
    y = xb[:, 0:128] * xb[:, 0:128]
    for k in range(1, E // 128):
        xk = xb[:, k * 128:(k + 1) * 128]
        y = y + xk * xk
    energy = jnp.sum(y, axis=1).reshape(1, SB)

    @pl.when(s == 0)
    def _():
        cs_ref[...] = colsum

    @pl.when(s != 0)
    def _():
        cs_ref[...] += colsum

    e_ref[pl.ds(b, 1), pl.ds(s * SB, SB)] = energy

    @pl.when(s == NSB - 1)
    def _():
        mean_ref[0] = cs_ref[...] * (1.0 / S)

    @pl.when((b == B - 1) & (s == NSB - 1))
    def _():
        lane_s = jax.lax.broadcasted_iota(jnp.int32, (B, S), 1)
        lane_w = jax.lax.broadcasted_iota(jnp.int32, (B, IDXW), 1)

        def body(j, carry):
            e, acc = carry
            m = jnp.max(e, axis=1, keepdims=True)
            idxv = jnp.min(jnp.where(e == m, lane_s, S), axis=1, keepdims=True)
            acc = jnp.where(lane_w == j, idxv, acc)
            e = jnp.where(lane_s == idxv, -1.0, e)
            return e, acc

        _, acc = jax.lax.fori_loop(
            0, SEL, body, (e_ref[...], jnp.zeros((B, IDXW), jnp.int32)))
        idx_ref[...] = acc.reshape(B, 1, IDXW)


def _stats(x):
    return pl.pallas_call(
        _stats_kernel,
        grid=(B, NSB),
        in_specs=[pl.BlockSpec((1, SB, E), lambda b, s: (b, s, 0))],
        out_specs=[
            pl.BlockSpec((1, 1, E), lambda b, s: (b, 0, 0)),
            pl.BlockSpec((B, 1, IDXW), lambda b, s: (0, 0, 0)),
        ],
        out_shape=[
            jax.ShapeDtypeStruct((B, 1, E), jnp.float32),
            jax.ShapeDtypeStruct((B, 1, IDXW), jnp.int32),
        ],
        scratch_shapes=[
            pltpu.VMEM((B, S), jnp.float32),
            pltpu.VMEM((1, E), jnp.float32),
        ],
    )(x)


def _bc(t):
    return jnp.minimum(t // NSB, B - 1)


def _bf(t):
    return jnp.maximum(t - NSB, 0) // NSB


def _sf(t):
    return jnp.maximum(t - NSB, 0) % NSB


def _mega_kernel(idxp_ref, xany_ref, x_ref, mean_ref, idx_ref,
                 wq_ref, wk_ref, wv_ref, wo_ref, bq_ref, bv_ref, bo_ref,
                 fill_ref,
                 qt_ref, xq_ref, rows_ref, acc_ref, m_ref, l_ref,
                 wq16_ref, wk16_ref, wv16_ref, wo16_ref, sem):
    t = pl.program_id(0)
    bc = _bc(t)
    s = t % NSB
    compute = t < B * NSB

    @pl.when(t == 0)
    def _():
        wq16_ref[...] = wq_ref[...].astype(BF)
        wk16_ref[...] = wk_ref[...].astype(BF)
        wv16_ref[...] = wv_ref[...].astype(BF)
        wo16_ref[...] = wo_ref[...].astype(BF)

    @pl.when(t >= NSB)
    def _():
        bf = _bf(t)
        sf = _sf(t)
        mean = mean_ref[0]
        delta = rows_ref[...] - mean
        row_g = jax.lax.broadcasted_iota(jnp.int32, (SB, SELP), 0) + sf * SB
        col_j = jax.lax.broadcasted_iota(jnp.int32, (SB, SELP), 1)
        idxv = idx_ref[bf, 0, :SELP].reshape(1, SELP)
        onehot = ((row_g == idxv) & (col_j < SEL)).astype(BF)
        fill_ref[0] = jnp.broadcast_to(mean, (SB, E)) + jax.lax.dot_general(
            onehot, delta.astype(BF), (((1,), (0,)), ((), ())),
            preferred_element_type=jnp.float32)

    @pl.when(compute & (s == 0))
    def _():
        copies = []
        for i in range(SELP):
            r = idxp_ref[bc, i]
            c = pltpu.make_async_copy(
                xany_ref.at[bc, pl.ds(r, 1), :], xq_ref.at[pl.ds(i, 1), :],
                sem)
            c.start()
            copies.append(c)
        for c in copies:
            c.wait()
        q = jax.lax.dot_general(
            xq_ref[...].astype(BF), wq16_ref[...], (((1,), (1,)), ((), ())),
            preferred_element_type=jnp.float32) + bq_ref[...]
        qe = jnp.concatenate([q] * H, axis=0)
        qe = (qe * _head_mask(ROWS, E, SELP, DH, jnp.float32)).astype(BF)
        qt = jax.lax.dot_general(
            qe, wk16_ref[...], (((1,), (0,)), ((), ())),
            preferred_element_type=jnp.float32)
        qt_ref[...] = (qt * SCALE).astype(BF)

    @pl.when(compute)
    def _():
        xb16 = x_ref[0].astype(BF)
        scores = jax.lax.dot_general(
            qt_ref[...], xb16, (((1,), (1,)), ((), ())),
            preferred_element_type=jnp.float32)
        bmax = jnp.max(scores, axis=1, keepdims=True)

        @pl.when(s == 0)
        def _():
            p = jnp.exp(scores - bmax)
            m_ref[...] = bmax
            l_ref[...] = jnp.sum(p, axis=1, keepdims=True)
            acc_ref[...] = jax.lax.dot_general(
                p.astype(BF), xb16, (((1,), (0,)), ((), ())),
                preferred_element_type=jnp.float32)

        @pl.when(s != 0)
        def _():
            m_old = m_ref[...]
            m_new = jnp.maximum(m_old, bmax)
            alpha = jnp.exp(m_old - m_new)
            p = jnp.exp(scores - m_new)
            m_ref[...] = m_new
            l_ref[...] = l_ref[...] * alpha + jnp.sum(p, axis=1, keepdims=True)
            acc_ref[...] = acc_ref[...] * alpha + jax.lax.dot_general(
                p.astype(BF), xb16, (((1,), (0,)), ((), ())),
                preferred_element_type=jnp.float32)

        @pl.when(s == NSB - 1)
        def _():
            z = acc_ref[...] / l_ref[...]
            oh = jax.lax.dot_general(
                z.astype(BF), wv16_ref[...], (((1,), (1,)), ((), ())),
                preferred_element_type=jnp.float32)
            oh = oh * _head_mask(ROWS, E, SELP, DH, jnp.float32)
            folded = jnp.zeros((SELP, E), jnp.float32)
            for h in range(H):
                folded = folded + oh[h * SELP:(h + 1) * SELP, :]
            bvo = jax.lax.dot_general(
                bv_ref[...].astype(BF), wo16_ref[...], (((1,), (1,)), ((), ())),
                preferred_element_type=jnp.float32) + bo_ref[...]
            rows_ref[...] = jax.lax.dot_general(
                folded.astype(BF), wo16_ref[...], (((1,), (1,)), ((), ())),
                preferred_element_type=jnp.float32) + bvo


def _mega(x, idx2d, mean, idx, W_qkv, W_out, bq, bv, bo):
    bspec = pl.BlockSpec((1, E), lambda t, p: (0, 0))
    return pl.pallas_call(
        _mega_kernel,
        grid_spec=pltpu.PrefetchScalarGridSpec(
            num_scalar_prefetch=1,
            grid=(T_STEPS,),
            in_specs=[
                pl.BlockSpec(memory_space=pl.ANY),
                pl.BlockSpec((1, SB, E),
                             lambda t, p: (_bc(t),
                                           jnp.minimum(t, B * NSB - 1) % NSB,
                                           0)),
                pl.BlockSpec((1, 1, E), lambda t, p: (_bf(t), 0, 0)),
                pl.BlockSpec((B, 1, IDXW), lambda t, p: (0, 0, 0)),
                pl.BlockSpec((E, E), lambda t, p: (0, 0)),
                pl.BlockSpec((E, E), lambda t, p: (1, 0)),
                pl.BlockSpec((E, E), lambda t, p: (2, 0)),
                pl.BlockSpec((E, E), lambda t, p: (0, 0)),
                bspec, bspec, bspec,
            ],
            out_specs=pl.BlockSpec(
                (1, SB, E), lambda t, p: (_bf(t), _sf(t), 0)),
            scratch_shapes=[
                pltpu.VMEM((ROWS, E), BF),
                pltpu.VMEM((SELP, E), jnp.float32),
                pltpu.VMEM((SELP, E), jnp.float32),
                pltpu.VMEM((ROWS, E), jnp.float32),
                pltpu.VMEM((ROWS, 1), jnp.float32),
                pltpu.VMEM((ROWS, 1), jnp.float32),
                pltpu.VMEM((E, E), BF),
                pltpu.VMEM((E, E), BF),
                pltpu.VMEM((E, E), BF),
                pltpu.VMEM((E, E), BF),
                pltpu.SemaphoreType.DMA,
            ],
        ),
        out_shape=jax.ShapeDtypeStruct((B, S, E), jnp.float32),
    )(idx2d, x, x, mean, idx, W_qkv, W_qkv, W_qkv, W_out, bq, bv, bo)


def kernel(x, W_qkv, b_qkv, W_out, b_out):
    bq = b_qkv[0:E].reshape(1, E)
    bv = b_qkv[2 * E:3 * E].reshape(1, E)
    bo = b_out.reshape(1, E)

    mean, idx = _stats(x)
    idx2d = idx.reshape(B, IDXW)
    return _mega(x, idx2d, mean, idx, W_qkv, W_out, bq, bv, bo)
